# Initial kernel scaffold; baseline (speedup 1.0000x reference)
import jax
import jax.numpy as jnp
from jax.experimental import pallas as pl


def kernel(row_embed, col_embed, h, w):
    raise NotImplementedError("write your pallas kernel here")


# speedup vs baseline: 3.0238x; 3.0238x over previous
import jax
import jax.numpy as jnp
from jax.experimental import pallas as pl

H = 384
W = 384
HALF = 256
D = 2 * HALF
BH = 8


def _body(row_ref, col_ref, out_ref):
    row = row_ref[...]
    col = col_ref[...]
    out_ref[:, :, :HALF] = jnp.broadcast_to(row[:, None, :], (BH, W, HALF))
    out_ref[:, :, HALF:] = jnp.broadcast_to(col[None, :, :], (BH, W, HALF))


def kernel(row_embed, col_embed, h, w):
    del h, w
    re = row_embed[:H]
    ce = col_embed[:W]
    return pl.pallas_call(
        _body,
        grid=(H // BH,),
        in_specs=[
            pl.BlockSpec((BH, HALF), lambda i: (i, 0)),
            pl.BlockSpec((W, HALF), lambda i: (0, 0)),
        ],
        out_specs=pl.BlockSpec((BH, W, D), lambda i: (i, 0, 0)),
        out_shape=jax.ShapeDtypeStruct((H, W, D), jnp.float32),
    )(re, ce)
